# baseline (speedup 1.0000x reference)
import functools
import jax
import jax.numpy as jnp
from jax import lax
from jax.experimental import pallas as pl
from jax.experimental.pallas import tpu as pltpu
from jax.experimental.pallas import tpu_sc as plsc

NC = 2
NS = 16
NW = NC * NS
B = 16384
K = 20
D = 64
BPW = B // NW
CHUNK = 16
NCHK = BPW // CHUNK
NROW = CHUNK * K

_LN2 = 0.6931471805599453


def _log_ge1(z):
    bits = plsc.bitcast(z, jnp.int32)
    e = lax.shift_right_arithmetic(bits, 23) - 127
    mbits = (bits & 0x7FFFFF) | 0x3F800000
    m = plsc.bitcast(mbits, jnp.float32)
    r = (m - 1.0) / (m + 1.0)
    r2 = r * r
    p = jnp.float32(1.0 / 9.0)
    p = p * r2 + jnp.float32(1.0 / 7.0)
    p = p * r2 + jnp.float32(1.0 / 5.0)
    p = p * r2 + jnp.float32(1.0 / 3.0)
    p = p * r2 + 1.0
    return e.astype(jnp.float32) * _LN2 + 2.0 * r * p


def _softplus(x):
    return _log_ge1(1.0 + jnp.exp(x))


def _sc_body(tgt_hbm, ctx_hbm, neg_hbm, wword_hbm, wctx_hbm, out_hbm,
             tgt_idx, ctx_idx, neg_idx, neg_flat, wbuf, cbuf, nbuf,
             shared, red, pvec, obuf, sem):
    c = lax.axis_index("c")
    s = lax.axis_index("s")
    wid = c * NS + s
    row0 = wid * BPW

    pltpu.sync_copy(tgt_hbm.at[pl.ds(row0, BPW)], tgt_idx)
    pltpu.sync_copy(ctx_hbm.at[pl.ds(row0, BPW)], ctx_idx)

    iota = lax.iota(jnp.int32, 16)
    rows_nb = iota * K

    HALF = BPW // 2
    for half in range(2):
        pltpu.sync_copy(
            neg_hbm.at[pl.ds(row0 + half * HALF, HALF), :], neg_idx)

        def flat_body(blk, _, half=half):
            rows = iota + blk * 16
            for k in range(K):
                colr = iota + k
                colr = jnp.where(colr >= K, colr - K, colr)
                v = plsc.load_gather(neg_idx, [rows, colr])
                plsc.store_scatter(
                    neg_flat, [(rows + half * HALF) * K + colr], v)
            return 0

        lax.fori_loop(0, HALF // 16, flat_body, 0)

    def copies(ci, p):
        cps = [
            pltpu.make_async_copy(
                wword_hbm.at[tgt_idx.at[pl.ds(ci * CHUNK, CHUNK)]],
                wbuf.at[p], sem.at[p]),
            pltpu.make_async_copy(
                wctx_hbm.at[ctx_idx.at[pl.ds(ci * CHUNK, CHUNK)]],
                cbuf.at[p], sem.at[p]),
        ]
        for off, ln in ((0, 128), (128, 128), (256, 64)):
            cps.append(pltpu.make_async_copy(
                wctx_hbm.at[neg_flat.at[pl.ds(ci * NROW + off, ln)]],
                nbuf.at[p, pl.ds(off, ln)], sem.at[p]))
        return cps

    for cp in copies(0, 0):
        cp.start()

    def chunk_body(ci, carry):
        pos_sp, neg_sp = carry
        p = lax.rem(ci, 2)

        @pl.when(ci + 1 < NCHK)
        def _():
            for cp in copies(ci + 1, 1 - p):
                cp.start()

        for cp in copies(ci, p):
            cp.wait()

        wb = wbuf.at[p]
        cb = cbuf.at[p]
        nb = nbuf.at[p]
        DUNROLL = 8

        def dblock(j, acc):
            acc = list(acc)
            for dd in range(DUNROLL):
                dcol = (iota + (j * DUNROLL + dd)) & (D - 1)
                wv = plsc.load_gather(wb, [iota, dcol])
                cv = plsc.load_gather(cb, [iota, dcol])
                acc[0] = acc[0] + wv * cv
                for k in range(K):
                    nv = plsc.load_gather(nb, [rows_nb + k, dcol])
                    acc[k + 1] = acc[k + 1] + nv * wv
            return tuple(acc)

        zeros = tuple(jnp.zeros((16,), jnp.float32) for _ in range(K + 1))
        acc = list(lax.fori_loop(0, D // DUNROLL, dblock, zeros))

        pos = jnp.clip(acc[0], -10.0, 10.0)
        pos_sp = pos_sp + _softplus(-pos)
        for k in range(K):
            ns = jnp.clip(acc[k + 1], -10.0, 10.0)
            neg_sp = neg_sp + _softplus(ns)
        return pos_sp, neg_sp

    zero = jnp.zeros((16,), jnp.float32)
    pos_sp, neg_sp = lax.fori_loop(0, NCHK, chunk_body, (zero, zero))

    pvec[...] = pos_sp * jnp.float32(1.0 / B) + neg_sp * jnp.float32(1.0 / (B * K))
    pltpu.sync_copy(pvec, shared.at[s])
    plsc.subcore_barrier()

    @pl.when(s == 0)
    def _():
        pltpu.sync_copy(shared, red)
        tot = jnp.zeros((16,), jnp.float32)
        for i in range(NS):
            tot = tot + red[i, :]
        obuf[...] = jnp.full((16,), jnp.sum(tot), jnp.float32)
        pltpu.sync_copy(obuf, out_hbm.at[c])


@jax.jit
def _sc_call(tgt, ctx, neg, W_word, W_ctx):
    mesh = plsc.VectorSubcoreMesh(
        core_axis_name="c", subcore_axis_name="s",
        num_cores=NC, num_subcores=NS)
    return pl.kernel(
        _sc_body,
        out_type=jax.ShapeDtypeStruct((NC, 16), jnp.float32),
        mesh=mesh,
        compiler_params=pltpu.CompilerParams(
            needs_layout_passes=False, use_tc_tiling_on_sc=False),
        scratch_types=[
            pltpu.VMEM((BPW,), jnp.int32),
            pltpu.VMEM((BPW,), jnp.int32),
            pltpu.VMEM((BPW // 2, 128), jnp.int32),
            pltpu.VMEM((BPW * K,), jnp.int32),
            pltpu.VMEM((2, CHUNK, D), jnp.float32),
            pltpu.VMEM((2, CHUNK, D), jnp.float32),
            pltpu.VMEM((2, NROW, D), jnp.float32),
            pltpu.VMEM_SHARED((NS, 16), jnp.float32),
            pltpu.VMEM((NS, 16), jnp.float32),
            pltpu.VMEM((16,), jnp.float32),
            pltpu.VMEM((16,), jnp.float32),
            pltpu.SemaphoreType.DMA((2,)),
        ],
    )(tgt, ctx, neg, W_word, W_ctx)


def kernel(target_word, context_word, negative_samples, W_word, W_ctx):
    tgt = target_word.astype(jnp.int32)
    ctx = context_word.astype(jnp.int32)
    neg = jnp.pad(negative_samples.astype(jnp.int32), ((0, 0), (0, 128 - K)))
    out = _sc_call(tgt, ctx, neg, W_word, W_ctx)
    return out[0, 0] + out[1, 0]


# speedup vs baseline: 1.0707x; 1.0707x over previous
import functools
import jax
import jax.numpy as jnp
from jax import lax
from jax.experimental import pallas as pl
from jax.experimental.pallas import tpu as pltpu
from jax.experimental.pallas import tpu_sc as plsc

NC = 2
NS = 16
NW = NC * NS
B = 16384
K = 20
D = 64
BPW = B // NW
CHUNK = 16
NCHK = BPW // CHUNK
NROW = CHUNK * K

_LN2 = 0.6931471805599453


def _log_ge1(z):
    bits = plsc.bitcast(z, jnp.int32)
    e = lax.shift_right_arithmetic(bits, 23) - 127
    mbits = (bits & 0x7FFFFF) | 0x3F800000
    m = plsc.bitcast(mbits, jnp.float32)
    r = (m - 1.0) / (m + 1.0)
    r2 = r * r
    p = jnp.float32(1.0 / 9.0)
    p = p * r2 + jnp.float32(1.0 / 7.0)
    p = p * r2 + jnp.float32(1.0 / 5.0)
    p = p * r2 + jnp.float32(1.0 / 3.0)
    p = p * r2 + 1.0
    return e.astype(jnp.float32) * _LN2 + 2.0 * r * p


def _softplus(x):
    return _log_ge1(1.0 + jnp.exp(x))


def _sc_body(tgt_hbm, ctx_hbm, neg_hbm, wword_hbm, wctx_hbm, out_hbm,
             tgt_idx, ctx_idx, neg_idx, neg_flat, wbuf, cbuf, nbuf,
             shared, red, pvec, obuf, sem):
    c = lax.axis_index("c")
    s = lax.axis_index("s")
    wid = c * NS + s
    row0 = wid * BPW

    pltpu.sync_copy(tgt_hbm.at[pl.ds(row0, BPW)], tgt_idx)
    pltpu.sync_copy(ctx_hbm.at[pl.ds(row0, BPW)], ctx_idx)

    iota = lax.iota(jnp.int32, 16)
    rows_nb = iota * K

    HALF = BPW // 2
    for half in range(2):
        pltpu.sync_copy(
            neg_hbm.at[pl.ds(row0 + half * HALF, HALF), :], neg_idx)

        def flat_body(blk, _, half=half):
            rows = iota + blk * 16
            for k in range(K):
                colr = iota + k
                colr = jnp.where(colr >= K, colr - K, colr)
                v = plsc.load_gather(neg_idx, [rows, colr])
                plsc.store_scatter(
                    neg_flat, [(rows + half * HALF) * K + colr], v)
            return 0

        lax.fori_loop(0, HALF // 16, flat_body, 0)

    def copies(ci, p):
        cps = [
            pltpu.make_async_copy(
                wword_hbm.at[tgt_idx.at[pl.ds(ci * CHUNK, CHUNK)]],
                wbuf.at[p], sem.at[p]),
            pltpu.make_async_copy(
                wctx_hbm.at[ctx_idx.at[pl.ds(ci * CHUNK, CHUNK)]],
                cbuf.at[p], sem.at[p]),
        ]
        for off, ln in ((0, 128), (128, 128), (256, 64)):
            cps.append(pltpu.make_async_copy(
                wctx_hbm.at[neg_flat.at[pl.ds(ci * NROW + off, ln)]],
                nbuf.at[p, pl.ds(off, ln)], sem.at[p]))
        return cps

    for cp in copies(0, 0):
        cp.start()

    def chunk_body(ci, carry):
        pos_sp, neg_sp = carry
        p = lax.rem(ci, 2)

        @pl.when(ci + 1 < NCHK)
        def _():
            for cp in copies(ci + 1, 1 - p):
                cp.start()

        for cp in copies(ci, p):
            cp.wait()

        wb = wbuf.at[p]
        cb = cbuf.at[p]
        nb = nbuf.at[p]
        DUNROLL = 16

        def dblock(j, acc):
            acc = list(acc)
            for dd in range(DUNROLL):
                dcol = (iota + (j * DUNROLL + dd)) & (D - 1)
                wv = plsc.load_gather(wb, [iota, dcol])
                cv = plsc.load_gather(cb, [iota, dcol])
                acc[0] = acc[0] + wv * cv
                for k in range(K):
                    nv = plsc.load_gather(nb, [rows_nb + k, dcol])
                    acc[k + 1] = acc[k + 1] + nv * wv
            return tuple(acc)

        zeros = tuple(jnp.zeros((16,), jnp.float32) for _ in range(K + 1))
        acc = list(lax.fori_loop(0, D // DUNROLL, dblock, zeros))

        pos = jnp.clip(acc[0], -10.0, 10.0)
        pos_sp = pos_sp + _softplus(-pos)
        for k in range(K):
            ns = jnp.clip(acc[k + 1], -10.0, 10.0)
            neg_sp = neg_sp + _softplus(ns)
        return pos_sp, neg_sp

    zero = jnp.zeros((16,), jnp.float32)
    pos_sp, neg_sp = lax.fori_loop(0, NCHK, chunk_body, (zero, zero))

    pvec[...] = pos_sp * jnp.float32(1.0 / B) + neg_sp * jnp.float32(1.0 / (B * K))
    pltpu.sync_copy(pvec, shared.at[s])
    plsc.subcore_barrier()

    @pl.when(s == 0)
    def _():
        pltpu.sync_copy(shared, red)
        tot = jnp.zeros((16,), jnp.float32)
        for i in range(NS):
            tot = tot + red[i, :]
        obuf[...] = jnp.full((16,), jnp.sum(tot), jnp.float32)
        pltpu.sync_copy(obuf, out_hbm.at[c])


@jax.jit
def _sc_call(tgt, ctx, neg, W_word, W_ctx):
    mesh = plsc.VectorSubcoreMesh(
        core_axis_name="c", subcore_axis_name="s",
        num_cores=NC, num_subcores=NS)
    return pl.kernel(
        _sc_body,
        out_type=jax.ShapeDtypeStruct((NC, 16), jnp.float32),
        mesh=mesh,
        compiler_params=pltpu.CompilerParams(
            needs_layout_passes=False, use_tc_tiling_on_sc=False),
        scratch_types=[
            pltpu.VMEM((BPW,), jnp.int32),
            pltpu.VMEM((BPW,), jnp.int32),
            pltpu.VMEM((BPW // 2, 128), jnp.int32),
            pltpu.VMEM((BPW * K,), jnp.int32),
            pltpu.VMEM((2, CHUNK, D), jnp.float32),
            pltpu.VMEM((2, CHUNK, D), jnp.float32),
            pltpu.VMEM((2, NROW, D), jnp.float32),
            pltpu.VMEM_SHARED((NS, 16), jnp.float32),
            pltpu.VMEM((NS, 16), jnp.float32),
            pltpu.VMEM((16,), jnp.float32),
            pltpu.VMEM((16,), jnp.float32),
            pltpu.SemaphoreType.DMA((2,)),
        ],
    )(tgt, ctx, neg, W_word, W_ctx)


def kernel(target_word, context_word, negative_samples, W_word, W_ctx):
    tgt = target_word.astype(jnp.int32)
    ctx = context_word.astype(jnp.int32)
    neg = jnp.pad(negative_samples.astype(jnp.int32), ((0, 0), (0, 128 - K)))
    out = _sc_call(tgt, ctx, neg, W_word, W_ctx)
    return out[0, 0] + out[1, 0]
